# Initial kernel scaffold; baseline (speedup 1.0000x reference)
import jax
import jax.numpy as jnp
from jax.experimental import pallas as pl


def kernel(user_emb, item_emb, user_mat, item_mat, edge_index, edge_weight):
    raise NotImplementedError("write your pallas kernel here")


# speedup vs baseline: 2.9244x; 2.9244x over previous
import functools

import jax
import jax.numpy as jnp
from jax import lax
from jax.experimental import pallas as pl
from jax.experimental.pallas import tpu as pltpu
from jax.experimental.pallas import tpu_sc as plsc

N = 50000
D = 64
DH = 32
H = 128
NC, NS, LANES = 2, 16, 16
E = 800000
CHUNK = 128
NCHUNK = 391
EPT = NCHUNK * CHUNK
EPAD = EPT * NS
RPT = 3136
NPAD = NS * RPT
WROWS = 784
WCHUNKS = RPT // WROWS

RB = 1000
GRID = N // RB


def _sc_body(dst_hbm, src_hbm, w_hbm, table_hbm, zeros_hbm, out_hbm,
             acc_sh, dst_v, src_v, idx_v, w_v, rows_v, out_v, sem):
    c = lax.axis_index("c")
    s = lax.axis_index("s")

    for q in range(WCHUNKS):
        pltpu.sync_copy(zeros_hbm,
                        acc_sh.at[pl.ds(s * RPT + q * WROWS, WROWS)])

    ebase = s * EPT

    def chunk_body(i, _):
        off = ebase + i * CHUNK
        pltpu.sync_copy(dst_hbm.at[pl.ds(off, CHUNK)], dst_v)
        pltpu.sync_copy(src_hbm.at[pl.ds(off, CHUNK)], src_v)
        pltpu.sync_copy(w_hbm.at[pl.ds(off, CHUNK)], w_v)
        coff = c * N

        def addoff(j, _):
            idx_v[pl.ds(j * LANES, LANES)] = (
                src_v[pl.ds(j * LANES, LANES)] + coff)
            return 0

        lax.fori_loop(0, CHUNK // LANES, addoff, 0, unroll=4)
        pltpu.async_copy(table_hbm.at[idx_v], rows_v, sem).wait()

        def scale(j, _):
            w16 = w_v[pl.ds(j * LANES, LANES)]
            for k in range(LANES):
                r = j * LANES + k
                wk = w16[k]
                rows_v[r, pl.ds(0, LANES)] = rows_v[r, pl.ds(0, LANES)] * wk
                rows_v[r, pl.ds(LANES, LANES)] = (
                    rows_v[r, pl.ds(LANES, LANES)] * wk)
            return 0

        lax.fori_loop(0, CHUNK // LANES, scale, 0)
        pltpu.sync_copy(rows_v, acc_sh.at[dst_v], add=True)
        return 0

    lax.fori_loop(0, NCHUNK, chunk_body, 0)

    plsc.subcore_barrier()

    def wchunk(q, _):
        r0 = s * RPT + q * WROWS
        pltpu.sync_copy(acc_sh.at[pl.ds(r0, WROWS)], out_v)

        def lk(r, _):
            v0 = out_v[r, pl.ds(0, LANES)]
            out_v[r, pl.ds(0, LANES)] = jnp.where(v0 >= 0.0, v0, 0.5 * v0)
            v1 = out_v[r, pl.ds(LANES, LANES)]
            out_v[r, pl.ds(LANES, LANES)] = jnp.where(v1 >= 0.0, v1, 0.5 * v1)
            return 0

        lax.fori_loop(0, WROWS, lk, 0, unroll=8)
        pltpu.sync_copy(out_v, out_hbm.at[c, pl.ds(r0, WROWS)])
        return 0

    lax.fori_loop(0, WCHUNKS, wchunk, 0)


@jax.jit
def _sc_spmm(dst, src, w, table_split):
    mesh = plsc.VectorSubcoreMesh(core_axis_name="c", subcore_axis_name="s")
    zeros = jnp.zeros((WROWS, DH), jnp.float32)
    f = pl.kernel(
        _sc_body,
        out_type=jax.ShapeDtypeStruct((NC, NPAD, DH), jnp.float32),
        mesh=mesh,
        scratch_types=[
            pltpu.VMEM_SHARED((NPAD, DH), jnp.float32),
            pltpu.VMEM((CHUNK,), jnp.int32),
            pltpu.VMEM((CHUNK,), jnp.int32),
            pltpu.VMEM((CHUNK,), jnp.int32),
            pltpu.VMEM((CHUNK,), jnp.float32),
            pltpu.VMEM((CHUNK, DH), jnp.float32),
            pltpu.VMEM((WROWS, DH), jnp.float32),
            pltpu.SemaphoreType.DMA,
        ],
        compiler_params=pltpu.CompilerParams(use_tc_tiling_on_sc=False),
    )
    return f(dst, src, w, table_split, zeros)[:, :N]


def _k01_body(u_ref, m_ref, hu_ref, he_ref):
    b = pl.program_id(0)
    u0 = u_ref[0]
    u1 = u_ref[1]
    m = m_ref[...]
    hu = (jnp.dot(u0, m[:DH], preferred_element_type=jnp.float32)
          + jnp.dot(u1, m[DH:], preferred_element_type=jnp.float32))
    hu_ref[...] = hu

    @pl.when(b == 0)
    def _():
        he_ref[...] = jnp.zeros_like(he_ref)

    he_ref[0] += lax.dot_general(hu, u0, (((0,), (0,)), ((), ())),
                                 preferred_element_type=jnp.float32)
    he_ref[1] += lax.dot_general(hu, u1, (((0,), (0,)), ((), ())),
                                 preferred_element_type=jnp.float32)


@jax.jit
def _k01(u_split, m):
    return pl.pallas_call(
        _k01_body,
        grid=(GRID,),
        in_specs=[
            pl.BlockSpec((NC, RB, DH), lambda b: (0, b, 0)),
            pl.BlockSpec((D, H), lambda b: (0, 0)),
        ],
        out_specs=[
            pl.BlockSpec((RB, H), lambda b: (b, 0)),
            pl.BlockSpec((NC, H, DH), lambda b: (0, 0, 0)),
        ],
        out_shape=[
            jax.ShapeDtypeStruct((N, H), jnp.float32),
            jax.ShapeDtypeStruct((NC, H, DH), jnp.float32),
        ],
    )(u_split, m)


def _k1_body(hu_ref, u_ref, he_ref):
    b = pl.program_id(0)
    hu = hu_ref[...]

    @pl.when(b == 0)
    def _():
        he_ref[...] = jnp.zeros_like(he_ref)

    he_ref[0] += lax.dot_general(hu, u_ref[0], (((0,), (0,)), ((), ())),
                                 preferred_element_type=jnp.float32)
    he_ref[1] += lax.dot_general(hu, u_ref[1], (((0,), (0,)), ((), ())),
                                 preferred_element_type=jnp.float32)


@jax.jit
def _k1(hu, u_split):
    return pl.pallas_call(
        _k1_body,
        grid=(GRID,),
        in_specs=[
            pl.BlockSpec((RB, H), lambda b: (b, 0)),
            pl.BlockSpec((NC, RB, DH), lambda b: (0, b, 0)),
        ],
        out_specs=pl.BlockSpec((NC, H, DH), lambda b: (0, 0, 0)),
        out_shape=jax.ShapeDtypeStruct((NC, H, DH), jnp.float32),
    )(hu, u_split)


def _k2a_body(hu_ref, he_ref, loc_ref, g_ref, u1_ref):
    hu = hu_ref[...]
    g0 = jnp.dot(hu, he_ref[0], preferred_element_type=jnp.float32)
    g1 = jnp.dot(hu, he_ref[1], preferred_element_type=jnp.float32)
    g_ref[0] = g0
    g_ref[1] = g1
    u1_ref[0] = loc_ref[0] + g0
    u1_ref[1] = loc_ref[1] + g1


@jax.jit
def _k2a(hu, he, loc_split):
    return pl.pallas_call(
        _k2a_body,
        grid=(GRID,),
        in_specs=[
            pl.BlockSpec((RB, H), lambda b: (b, 0)),
            pl.BlockSpec((NC, H, DH), lambda b: (0, 0, 0)),
            pl.BlockSpec((NC, RB, DH), lambda b: (0, b, 0)),
        ],
        out_specs=[
            pl.BlockSpec((NC, RB, DH), lambda b: (0, b, 0)),
            pl.BlockSpec((NC, RB, DH), lambda b: (0, b, 0)),
        ],
        out_shape=[
            jax.ShapeDtypeStruct((NC, N, DH), jnp.float32),
            jax.ShapeDtypeStruct((NC, N, DH), jnp.float32),
        ],
    )(hu, he, loc_split)


def _k2b_body(hu_ref, he_ref, loc_ref, u0_ref, l1_ref, g1_ref,
              g_ref, fin_ref):
    hu = hu_ref[...]
    g0 = jnp.dot(hu, he_ref[0], preferred_element_type=jnp.float32)
    g1 = jnp.dot(hu, he_ref[1], preferred_element_type=jnp.float32)
    g_ref[0] = g0
    g_ref[1] = g1
    fin_ref[0] = u0_ref[0] + l1_ref[0] + g1_ref[0] + loc_ref[0] + g0
    fin_ref[1] = u0_ref[1] + l1_ref[1] + g1_ref[1] + loc_ref[1] + g1


@jax.jit
def _k2b(hu, he, loc_split, u0_split, l1_split, g1_split):
    spec = pl.BlockSpec((NC, RB, DH), lambda b: (0, b, 0))
    return pl.pallas_call(
        _k2b_body,
        grid=(GRID,),
        in_specs=[
            pl.BlockSpec((RB, H), lambda b: (b, 0)),
            pl.BlockSpec((NC, H, DH), lambda b: (0, 0, 0)),
            spec, spec, spec, spec,
        ],
        out_specs=[spec, spec],
        out_shape=[
            jax.ShapeDtypeStruct((NC, N, DH), jnp.float32),
            jax.ShapeDtypeStruct((NC, N, DH), jnp.float32),
        ],
    )(hu, he, loc_split, u0_split, l1_split, g1_split)


def _split(x):
    return x.reshape(N, NC, DH).transpose(1, 0, 2)


def _unsplit(x):
    return x.transpose(1, 0, 2).reshape(N, D)


def kernel(user_emb, item_emb, user_mat, item_mat, edge_index, edge_weight):
    row = edge_index[0].astype(jnp.int32)
    col = edge_index[1].astype(jnp.int32)
    pad_i = jnp.zeros((EPAD - E,), jnp.int32)
    pad_f = jnp.zeros((EPAD - E,), jnp.float32)
    row_p = jnp.concatenate([row, pad_i])
    col_p = jnp.concatenate([col, pad_i])
    w_p = jnp.concatenate([edge_weight, pad_f])

    u0 = _split(user_emb)
    i0 = _split(item_emb)
    u0_flat = u0.reshape(NC * N, DH)
    i0_flat = i0.reshape(NC * N, DH)

    hu, he1u = _k01(u0, user_mat)
    hi, he1i = _k01(i0, item_mat)

    lu1 = _sc_spmm(row_p, col_p, w_p, i0_flat)
    li1 = _sc_spmm(col_p, row_p, w_p, u0_flat)

    gu1, u1 = _k2a(hu, he1u, lu1)
    gi1, i1 = _k2a(hi, he1i, li1)

    he2u = _k1(hu, u1)
    he2i = _k1(hi, i1)

    lu2 = _sc_spmm(row_p, col_p, w_p, i1.reshape(NC * N, DH))
    li2 = _sc_spmm(col_p, row_p, w_p, u1.reshape(NC * N, DH))

    gu2, fin_u = _k2b(hu, he2u, lu2, u0, lu1, gu1)
    gi2, fin_i = _k2b(hi, he2i, li2, i0, li1, gi1)

    return (_unsplit(fin_u), _unsplit(fin_i),
            _unsplit(lu1), _unsplit(lu2),
            _unsplit(li1), _unsplit(li2),
            _unsplit(gu1), _unsplit(gu2),
            _unsplit(gi1), _unsplit(gi2))


# speedup vs baseline: 7.4545x; 2.5490x over previous
import functools

import jax
import jax.numpy as jnp
from jax import lax
from jax.experimental import pallas as pl
from jax.experimental.pallas import tpu as pltpu
from jax.experimental.pallas import tpu_sc as plsc

N = 50000
D = 64
DH = 32
H = 128
NC, NS, LANES = 2, 16, 16
E = 800000
CHUNK = 128
SROWS = 4
NSUP = 98
EPT = SROWS * CHUNK * NSUP
EPAD = EPT * NS
RPT = 3136
NPAD = NS * RPT
WROWS = 224
WCHUNKS = RPT // WROWS

RB = 1000
GRID = N // RB


def _sc_body(dst_hbm, src_hbm, w_hbm, table_hbm, zeros_hbm, out_hbm,
             acc_sh, dst_v, src_v, w_v, idx_v, rows_v, out_v, gsem, ssem, psem):
    c = lax.axis_index("c")
    s = lax.axis_index("s")

    for q in range(WCHUNKS):
        pltpu.sync_copy(zeros_hbm,
                        acc_sh.at[pl.ds(s * RPT + q * WROWS, WROWS)])

    tbase = s * SROWS * NSUP

    def prefetch(sup, slot):
        r0 = tbase + sup * SROWS
        pltpu.async_copy(dst_hbm.at[pl.ds(r0, SROWS)], dst_v.at[slot], psem)
        pltpu.async_copy(src_hbm.at[pl.ds(r0, SROWS)], src_v.at[slot], psem)
        pltpu.async_copy(w_hbm.at[pl.ds(r0, SROWS)], w_v.at[slot], psem)

    def drain_prefetch(slot):
        pltpu.make_async_copy(dst_hbm.at[pl.ds(0, SROWS)], dst_v.at[slot],
                              psem).wait()
        pltpu.make_async_copy(src_hbm.at[pl.ds(0, SROWS)], src_v.at[slot],
                              psem).wait()
        pltpu.make_async_copy(w_hbm.at[pl.ds(0, SROWS)], w_v.at[slot],
                              psem).wait()

    prefetch(0, 0)
    coff = c * N

    def super_body(sup, _):
        slot = lax.rem(sup, 2)
        drain_prefetch(slot)

        @pl.when(sup + 1 < NSUP)
        def _():
            prefetch(sup + 1, 1 - slot)

        def addoff(j, _):
            r = j // (CHUNK // LANES)
            l = j % (CHUNK // LANES)
            idx_v[r, pl.ds(l * LANES, LANES)] = (
                src_v[slot, r, pl.ds(l * LANES, LANES)] + coff)
            return 0

        lax.fori_loop(0, SROWS * (CHUNK // LANES), addoff, 0, unroll=8)

        for j in range(SROWS):
            pltpu.async_copy(table_hbm.at[idx_v.at[j]], rows_v.at[j], gsem)

        for j in range(SROWS):
            pltpu.make_async_copy(table_hbm.at[idx_v.at[j]], rows_v.at[j],
                                  gsem).wait()

            def scale(l, _):
                w16 = w_v[slot, j, pl.ds(l * LANES, LANES)]
                for k in range(LANES):
                    r = l * LANES + k
                    wk = w16[k]
                    rows_v[j, r, pl.ds(0, LANES)] = (
                        rows_v[j, r, pl.ds(0, LANES)] * wk)
                    rows_v[j, r, pl.ds(LANES, LANES)] = (
                        rows_v[j, r, pl.ds(LANES, LANES)] * wk)
                return 0

            lax.fori_loop(0, CHUNK // LANES, scale, 0)
            pltpu.async_copy(rows_v.at[j], acc_sh.at[dst_v.at[slot, j]],
                             ssem, add=True)

        for j in range(SROWS):
            pltpu.make_async_copy(rows_v.at[j], acc_sh.at[dst_v.at[slot, j]],
                                  ssem).wait()
        return 0

    lax.fori_loop(0, NSUP, super_body, 0)

    plsc.subcore_barrier()

    def wchunk(q, _):
        r0 = s * RPT + q * WROWS
        pltpu.sync_copy(acc_sh.at[pl.ds(r0, WROWS)], out_v)

        def lk(r, _):
            v0 = out_v[r, pl.ds(0, LANES)]
            out_v[r, pl.ds(0, LANES)] = jnp.where(v0 >= 0.0, v0, 0.5 * v0)
            v1 = out_v[r, pl.ds(LANES, LANES)]
            out_v[r, pl.ds(LANES, LANES)] = jnp.where(v1 >= 0.0, v1, 0.5 * v1)
            return 0

        lax.fori_loop(0, WROWS, lk, 0, unroll=8)
        pltpu.sync_copy(out_v, out_hbm.at[c, pl.ds(r0, WROWS)])
        return 0

    lax.fori_loop(0, WCHUNKS, wchunk, 0)


@jax.jit
def _sc_spmm(dst, src, w, table_split):
    mesh = plsc.VectorSubcoreMesh(core_axis_name="c", subcore_axis_name="s")
    zeros = jnp.zeros((WROWS, DH), jnp.float32)
    f = pl.kernel(
        _sc_body,
        out_type=jax.ShapeDtypeStruct((NC, NPAD, DH), jnp.float32),
        mesh=mesh,
        scratch_types=[
            pltpu.VMEM_SHARED((NPAD, DH), jnp.float32),
            pltpu.VMEM((2, SROWS, CHUNK), jnp.int32),
            pltpu.VMEM((2, SROWS, CHUNK), jnp.int32),
            pltpu.VMEM((2, SROWS, CHUNK), jnp.float32),
            pltpu.VMEM((SROWS, CHUNK), jnp.int32),
            pltpu.VMEM((SROWS, CHUNK, DH), jnp.float32),
            pltpu.VMEM((WROWS, DH), jnp.float32),
            pltpu.SemaphoreType.DMA,
            pltpu.SemaphoreType.DMA,
            pltpu.SemaphoreType.DMA,
        ],
        compiler_params=pltpu.CompilerParams(use_tc_tiling_on_sc=False),
    )
    d2 = dst.reshape(EPAD // CHUNK, CHUNK)
    s2 = src.reshape(EPAD // CHUNK, CHUNK)
    w2 = w.reshape(EPAD // CHUNK, CHUNK)
    return f(d2, s2, w2, table_split, zeros)[:, :N]


def _k01_body(u_ref, m_ref, hu_ref, he_ref):
    b = pl.program_id(0)
    u0 = u_ref[0]
    u1 = u_ref[1]
    m = m_ref[...]
    hu = (jnp.dot(u0, m[:DH], preferred_element_type=jnp.float32)
          + jnp.dot(u1, m[DH:], preferred_element_type=jnp.float32))
    hu_ref[...] = hu

    @pl.when(b == 0)
    def _():
        he_ref[...] = jnp.zeros_like(he_ref)

    he_ref[0] += lax.dot_general(hu, u0, (((0,), (0,)), ((), ())),
                                 preferred_element_type=jnp.float32)
    he_ref[1] += lax.dot_general(hu, u1, (((0,), (0,)), ((), ())),
                                 preferred_element_type=jnp.float32)


@jax.jit
def _k01(u_split, m):
    return pl.pallas_call(
        _k01_body,
        grid=(GRID,),
        in_specs=[
            pl.BlockSpec((NC, RB, DH), lambda b: (0, b, 0)),
            pl.BlockSpec((D, H), lambda b: (0, 0)),
        ],
        out_specs=[
            pl.BlockSpec((RB, H), lambda b: (b, 0)),
            pl.BlockSpec((NC, H, DH), lambda b: (0, 0, 0)),
        ],
        out_shape=[
            jax.ShapeDtypeStruct((N, H), jnp.float32),
            jax.ShapeDtypeStruct((NC, H, DH), jnp.float32),
        ],
    )(u_split, m)


def _k1_body(hu_ref, u_ref, he_ref):
    b = pl.program_id(0)
    hu = hu_ref[...]

    @pl.when(b == 0)
    def _():
        he_ref[...] = jnp.zeros_like(he_ref)

    he_ref[0] += lax.dot_general(hu, u_ref[0], (((0,), (0,)), ((), ())),
                                 preferred_element_type=jnp.float32)
    he_ref[1] += lax.dot_general(hu, u_ref[1], (((0,), (0,)), ((), ())),
                                 preferred_element_type=jnp.float32)


@jax.jit
def _k1(hu, u_split):
    return pl.pallas_call(
        _k1_body,
        grid=(GRID,),
        in_specs=[
            pl.BlockSpec((RB, H), lambda b: (b, 0)),
            pl.BlockSpec((NC, RB, DH), lambda b: (0, b, 0)),
        ],
        out_specs=pl.BlockSpec((NC, H, DH), lambda b: (0, 0, 0)),
        out_shape=jax.ShapeDtypeStruct((NC, H, DH), jnp.float32),
    )(hu, u_split)


def _k2a_body(hu_ref, he_ref, loc_ref, g_ref, u1_ref):
    hu = hu_ref[...]
    g0 = jnp.dot(hu, he_ref[0], preferred_element_type=jnp.float32)
    g1 = jnp.dot(hu, he_ref[1], preferred_element_type=jnp.float32)
    g_ref[0] = g0
    g_ref[1] = g1
    u1_ref[0] = loc_ref[0] + g0
    u1_ref[1] = loc_ref[1] + g1


@jax.jit
def _k2a(hu, he, loc_split):
    return pl.pallas_call(
        _k2a_body,
        grid=(GRID,),
        in_specs=[
            pl.BlockSpec((RB, H), lambda b: (b, 0)),
            pl.BlockSpec((NC, H, DH), lambda b: (0, 0, 0)),
            pl.BlockSpec((NC, RB, DH), lambda b: (0, b, 0)),
        ],
        out_specs=[
            pl.BlockSpec((NC, RB, DH), lambda b: (0, b, 0)),
            pl.BlockSpec((NC, RB, DH), lambda b: (0, b, 0)),
        ],
        out_shape=[
            jax.ShapeDtypeStruct((NC, N, DH), jnp.float32),
            jax.ShapeDtypeStruct((NC, N, DH), jnp.float32),
        ],
    )(hu, he, loc_split)


def _k2b_body(hu_ref, he_ref, loc_ref, u0_ref, l1_ref, g1_ref,
              g_ref, fin_ref):
    hu = hu_ref[...]
    g0 = jnp.dot(hu, he_ref[0], preferred_element_type=jnp.float32)
    g1 = jnp.dot(hu, he_ref[1], preferred_element_type=jnp.float32)
    g_ref[0] = g0
    g_ref[1] = g1
    fin_ref[0] = u0_ref[0] + l1_ref[0] + g1_ref[0] + loc_ref[0] + g0
    fin_ref[1] = u0_ref[1] + l1_ref[1] + g1_ref[1] + loc_ref[1] + g1


@jax.jit
def _k2b(hu, he, loc_split, u0_split, l1_split, g1_split):
    spec = pl.BlockSpec((NC, RB, DH), lambda b: (0, b, 0))
    return pl.pallas_call(
        _k2b_body,
        grid=(GRID,),
        in_specs=[
            pl.BlockSpec((RB, H), lambda b: (b, 0)),
            pl.BlockSpec((NC, H, DH), lambda b: (0, 0, 0)),
            spec, spec, spec, spec,
        ],
        out_specs=[spec, spec],
        out_shape=[
            jax.ShapeDtypeStruct((NC, N, DH), jnp.float32),
            jax.ShapeDtypeStruct((NC, N, DH), jnp.float32),
        ],
    )(hu, he, loc_split, u0_split, l1_split, g1_split)


def _split(x):
    return x.reshape(N, NC, DH).transpose(1, 0, 2)


def _unsplit(x):
    return x.transpose(1, 0, 2).reshape(N, D)


def kernel(user_emb, item_emb, user_mat, item_mat, edge_index, edge_weight):
    row = edge_index[0].astype(jnp.int32)
    col = edge_index[1].astype(jnp.int32)
    pad_i = jnp.zeros((EPAD - E,), jnp.int32)
    pad_f = jnp.zeros((EPAD - E,), jnp.float32)
    row_p = jnp.concatenate([row, pad_i])
    col_p = jnp.concatenate([col, pad_i])
    w_p = jnp.concatenate([edge_weight, pad_f])

    u0 = _split(user_emb)
    i0 = _split(item_emb)
    u0_flat = u0.reshape(NC * N, DH)
    i0_flat = i0.reshape(NC * N, DH)

    hu, he1u = _k01(u0, user_mat)
    hi, he1i = _k01(i0, item_mat)

    lu1 = _sc_spmm(row_p, col_p, w_p, i0_flat)
    li1 = _sc_spmm(col_p, row_p, w_p, u0_flat)

    gu1, u1 = _k2a(hu, he1u, lu1)
    gi1, i1 = _k2a(hi, he1i, li1)

    he2u = _k1(hu, u1)
    he2i = _k1(hi, i1)

    lu2 = _sc_spmm(row_p, col_p, w_p, i1.reshape(NC * N, DH))
    li2 = _sc_spmm(col_p, row_p, w_p, u1.reshape(NC * N, DH))

    gu2, fin_u = _k2b(hu, he2u, lu2, u0, lu1, gu1)
    gi2, fin_i = _k2b(hi, he2i, li2, i0, li1, gi1)

    return (_unsplit(fin_u), _unsplit(fin_i),
            _unsplit(lu1), _unsplit(lu2),
            _unsplit(li1), _unsplit(li2),
            _unsplit(gu1), _unsplit(gu2),
            _unsplit(gi1), _unsplit(gi2))


# speedup vs baseline: 8.4070x; 1.1278x over previous
import jax
import jax.numpy as jnp
from jax import lax
from jax.experimental import pallas as pl
from jax.experimental.pallas import tpu as pltpu
from jax.experimental.pallas import tpu_sc as plsc

N = 50000
D = 64
DH = 32
H = 128
NC, NS, LANES = 2, 16, 16
E = 800000
CHUNK = 128
SROWS = 4
NSUP = 98
EPT = SROWS * CHUNK * NSUP
EPAD = EPT * NS
RPT = N // NS
WROWS = 125
WCHUNKS = RPT // WROWS

RB = 1000
GRID = N // RB


def _sc_body(dst_hbm, src_hbm, w_hbm, table_hbm, zeros_hbm, out_hbm,
             acc_sh, dst_v, src_v, w_v, idx_v, rows_v, out_v,
             gsem, ssem, psem):
    c = lax.axis_index("c")
    s = lax.axis_index("s")

    for q in range(WCHUNKS):
        pltpu.sync_copy(zeros_hbm,
                        acc_sh.at[pl.ds(s * RPT + q * WROWS, WROWS)])

    tbase = s * SROWS * NSUP

    def prefetch(sup, slot):
        r0 = tbase + sup * SROWS
        pltpu.async_copy(dst_hbm.at[pl.ds(r0, SROWS)], dst_v.at[slot], psem)
        pltpu.async_copy(src_hbm.at[pl.ds(r0, SROWS)], src_v.at[slot], psem)
        pltpu.async_copy(w_hbm.at[pl.ds(r0, SROWS)], w_v.at[slot], psem)

    def drain_prefetch(slot):
        pltpu.make_async_copy(dst_hbm.at[pl.ds(0, SROWS)], dst_v.at[slot],
                              psem).wait()
        pltpu.make_async_copy(src_hbm.at[pl.ds(0, SROWS)], src_v.at[slot],
                              psem).wait()
        pltpu.make_async_copy(w_hbm.at[pl.ds(0, SROWS)], w_v.at[slot],
                              psem).wait()

    prefetch(0, 0)
    coff = c * N

    def super_body(sup, _):
        slot = lax.rem(sup, 2)
        drain_prefetch(slot)

        @pl.when(sup + 1 < NSUP)
        def _():
            prefetch(sup + 1, 1 - slot)

        def addoff(j, _):
            r = j // (CHUNK // LANES)
            l = j % (CHUNK // LANES)
            idx_v[r, pl.ds(l * LANES, LANES)] = (
                src_v[slot, r, pl.ds(l * LANES, LANES)] + coff)
            return 0

        lax.fori_loop(0, SROWS * (CHUNK // LANES), addoff, 0, unroll=8)

        for j in range(SROWS):
            pltpu.async_copy(table_hbm.at[idx_v.at[j]], rows_v.at[j], gsem)

        for j in range(SROWS):
            pltpu.make_async_copy(table_hbm.at[idx_v.at[j]], rows_v.at[j],
                                  gsem).wait()

            def scale(l, _):
                w16 = w_v[slot, j, pl.ds(l * LANES, LANES)]
                for k in range(LANES):
                    r = l * LANES + k
                    wk = w16[k]
                    rows_v[j, r, pl.ds(0, LANES)] = (
                        rows_v[j, r, pl.ds(0, LANES)] * wk)
                    rows_v[j, r, pl.ds(LANES, LANES)] = (
                        rows_v[j, r, pl.ds(LANES, LANES)] * wk)
                return 0

            lax.fori_loop(0, CHUNK // LANES, scale, 0)
            pltpu.async_copy(rows_v.at[j], acc_sh.at[dst_v.at[slot, j]],
                             ssem, add=True)

        for j in range(SROWS):
            pltpu.make_async_copy(rows_v.at[j], acc_sh.at[dst_v.at[slot, j]],
                                  ssem).wait()
        return 0

    lax.fori_loop(0, NSUP, super_body, 0)

    plsc.subcore_barrier()

    def wchunk(q, _):
        r0 = s * RPT + q * WROWS
        pltpu.sync_copy(acc_sh.at[pl.ds(r0, WROWS)], out_v)

        def lk(r, _):
            v0 = out_v[r, pl.ds(0, LANES)]
            out_v[r, pl.ds(0, LANES)] = jnp.where(v0 >= 0.0, v0, 0.5 * v0)
            v1 = out_v[r, pl.ds(LANES, LANES)]
            out_v[r, pl.ds(LANES, LANES)] = jnp.where(v1 >= 0.0, v1, 0.5 * v1)
            return 0

        lax.fori_loop(0, WROWS, lk, 0, unroll=8)
        pltpu.sync_copy(out_v, out_hbm.at[pl.ds(r0, WROWS),
                                          pl.ds(c * DH, DH)])
        return 0

    lax.fori_loop(0, WCHUNKS, wchunk, 0)


@jax.jit
def _sc_spmm(dst, src, w, table_split):
    mesh = plsc.VectorSubcoreMesh(core_axis_name="c", subcore_axis_name="s")
    zeros = jnp.zeros((WROWS, DH), jnp.float32)
    f = pl.kernel(
        _sc_body,
        out_type=jax.ShapeDtypeStruct((N, D), jnp.float32),
        mesh=mesh,
        scratch_types=[
            pltpu.VMEM_SHARED((N, DH), jnp.float32),
            pltpu.VMEM((2, SROWS, CHUNK), jnp.int32),
            pltpu.VMEM((2, SROWS, CHUNK), jnp.int32),
            pltpu.VMEM((2, SROWS, CHUNK), jnp.float32),
            pltpu.VMEM((SROWS, CHUNK), jnp.int32),
            pltpu.VMEM((SROWS, CHUNK, DH), jnp.float32),
            pltpu.VMEM((WROWS, DH), jnp.float32),
            pltpu.SemaphoreType.DMA,
            pltpu.SemaphoreType.DMA,
            pltpu.SemaphoreType.DMA,
        ],
        compiler_params=pltpu.CompilerParams(use_tc_tiling_on_sc=False),
    )
    d2 = dst.reshape(EPAD // CHUNK, CHUNK)
    s2 = src.reshape(EPAD // CHUNK, CHUNK)
    w2 = w.reshape(EPAD // CHUNK, CHUNK)
    return f(d2, s2, w2, table_split, zeros)


def _k01_body(u_ref, m_ref, hu_ref, he_ref):
    b = pl.program_id(0)
    u = u_ref[...]
    hu = jnp.dot(u, m_ref[...], preferred_element_type=jnp.float32)
    hu_ref[...] = hu

    @pl.when(b == 0)
    def _():
        he_ref[...] = jnp.zeros_like(he_ref)

    he_ref[...] += lax.dot_general(hu, u, (((0,), (0,)), ((), ())),
                                   preferred_element_type=jnp.float32)


@jax.jit
def _k01(u, m):
    return pl.pallas_call(
        _k01_body,
        grid=(GRID,),
        in_specs=[
            pl.BlockSpec((RB, D), lambda b: (b, 0)),
            pl.BlockSpec((D, H), lambda b: (0, 0)),
        ],
        out_specs=[
            pl.BlockSpec((RB, H), lambda b: (b, 0)),
            pl.BlockSpec((H, D), lambda b: (0, 0)),
        ],
        out_shape=[
            jax.ShapeDtypeStruct((N, H), jnp.float32),
            jax.ShapeDtypeStruct((H, D), jnp.float32),
        ],
    )(u, m)


def _k1_body(hu_ref, u_ref, he_ref):
    b = pl.program_id(0)
    hu = hu_ref[...]
    u = jnp.concatenate([u_ref[0], u_ref[1]], axis=1)

    @pl.when(b == 0)
    def _():
        he_ref[...] = jnp.zeros_like(he_ref)

    he_ref[...] += lax.dot_general(hu, u, (((0,), (0,)), ((), ())),
                                   preferred_element_type=jnp.float32)


@jax.jit
def _k1(hu, u_split):
    return pl.pallas_call(
        _k1_body,
        grid=(GRID,),
        in_specs=[
            pl.BlockSpec((RB, H), lambda b: (b, 0)),
            pl.BlockSpec((NC, RB, DH), lambda b: (0, b, 0)),
        ],
        out_specs=pl.BlockSpec((H, D), lambda b: (0, 0)),
        out_shape=jax.ShapeDtypeStruct((H, D), jnp.float32),
    )(hu, u_split)


def _k2a_body(hu_ref, he_ref, loc_ref, g_ref, ucat_ref):
    g = jnp.dot(hu_ref[...], he_ref[...], preferred_element_type=jnp.float32)
    g_ref[...] = g
    u1 = loc_ref[...] + g
    ucat_ref[0] = u1[:, :DH]
    ucat_ref[1] = u1[:, DH:]


@jax.jit
def _k2a(hu, he, loc):
    return pl.pallas_call(
        _k2a_body,
        grid=(GRID,),
        in_specs=[
            pl.BlockSpec((RB, H), lambda b: (b, 0)),
            pl.BlockSpec((H, D), lambda b: (0, 0)),
            pl.BlockSpec((RB, D), lambda b: (b, 0)),
        ],
        out_specs=[
            pl.BlockSpec((RB, D), lambda b: (b, 0)),
            pl.BlockSpec((NC, RB, DH), lambda b: (0, b, 0)),
        ],
        out_shape=[
            jax.ShapeDtypeStruct((N, D), jnp.float32),
            jax.ShapeDtypeStruct((NC, N, DH), jnp.float32),
        ],
    )(hu, he, loc)


def _k2b_body(hu_ref, he_ref, loc_ref, u0_ref, l1_ref, g1_ref,
              g_ref, fin_ref):
    g = jnp.dot(hu_ref[...], he_ref[...], preferred_element_type=jnp.float32)
    g_ref[...] = g
    fin_ref[...] = (u0_ref[...] + l1_ref[...] + g1_ref[...]
                    + loc_ref[...] + g)


@jax.jit
def _k2b(hu, he, loc, u0, l1, g1):
    spec = pl.BlockSpec((RB, D), lambda b: (b, 0))
    return pl.pallas_call(
        _k2b_body,
        grid=(GRID,),
        in_specs=[
            pl.BlockSpec((RB, H), lambda b: (b, 0)),
            pl.BlockSpec((H, D), lambda b: (0, 0)),
            spec, spec, spec, spec,
        ],
        out_specs=[spec, spec],
        out_shape=[
            jax.ShapeDtypeStruct((N, D), jnp.float32),
            jax.ShapeDtypeStruct((N, D), jnp.float32),
        ],
    )(hu, he, loc, u0, l1, g1)


def _split(x):
    return x.reshape(N, NC, DH).transpose(1, 0, 2)


def kernel(user_emb, item_emb, user_mat, item_mat, edge_index, edge_weight):
    row = edge_index[0].astype(jnp.int32)
    col = edge_index[1].astype(jnp.int32)
    pad_i = jnp.zeros((EPAD - E,), jnp.int32)
    pad_f = jnp.zeros((EPAD - E,), jnp.float32)
    row_p = jnp.concatenate([row, pad_i])
    col_p = jnp.concatenate([col, pad_i])
    w_p = jnp.concatenate([edge_weight, pad_f])

    u0c = _split(user_emb)
    i0c = _split(item_emb)
    u0_flat = u0c.reshape(NC * N, DH)
    i0_flat = i0c.reshape(NC * N, DH)

    hu, he1u = _k01(user_emb, user_mat)
    hi, he1i = _k01(item_emb, item_mat)

    lu1 = _sc_spmm(row_p, col_p, w_p, i0_flat)
    li1 = _sc_spmm(col_p, row_p, w_p, u0_flat)

    gu1, u1c = _k2a(hu, he1u, lu1)
    gi1, i1c = _k2a(hi, he1i, li1)

    he2u = _k1(hu, u1c)
    he2i = _k1(hi, i1c)

    lu2 = _sc_spmm(row_p, col_p, w_p, i1c.reshape(NC * N, DH))
    li2 = _sc_spmm(col_p, row_p, w_p, u1c.reshape(NC * N, DH))

    gu2, fin_u = _k2b(hu, he2u, lu2, user_emb, lu1, gu1)
    gi2, fin_i = _k2b(hi, he2i, li2, item_emb, li1, gi1)

    return (fin_u, fin_i, lu1, lu2, li1, li2, gu1, gu2, gi1, gi2)
